# baseline (speedup 1.0000x reference)
import dataclasses

import jax
import jax.numpy as jnp
from jax import lax
from jax.experimental import pallas as pl
from jax.experimental.pallas import tpu as pltpu
from jax.experimental.pallas import tpu_sc as plsc

EMBED_DIM = 128
GAZ_DIM = 64
OUT_DIM = EMBED_DIM + GAZ_DIM
WINDOW = 128
LANES = 16


def _embed_concat(sentence_data, gazetteers_data, word_table):
    num_tokens = sentence_data.shape[0]
    nw = num_tokens // WINDOW
    idx2d = sentence_data.reshape(nw, WINDOW)
    gaz_t3 = gazetteers_data.T.reshape(GAZ_DIM // 8, 8, num_tokens)
    mesh = plsc.VectorSubcoreMesh(core_axis_name="core",
                                  subcore_axis_name="subcore")

    cp = pltpu.CompilerParams()
    if "needs_layout_passes" in pltpu.CompilerParams.__dataclass_fields__:
        cp = dataclasses.replace(cp, needs_layout_passes=False)

    @pl.kernel(
        out_type=jax.ShapeDtypeStruct((OUT_DIM // 8, nw, 8, WINDOW),
                                      jnp.float32),
        mesh=mesh,
        compiler_params=cp,
        scratch_types=[pltpu.VMEM((WINDOW, EMBED_DIM), jnp.float32),
                       pltpu.SemaphoreType.DMA,
                       pltpu.SemaphoreType.DMA],
    )
    def kern(idx_hbm, gazt_hbm, table_hbm, out_hbm, scr, gsem, zsem):
        def body(indices, i_vmem, o_vmem):
            (w,) = indices
            zcp = pltpu.async_copy(
                gazt_hbm.at[:, :, pl.ds(w * WINDOW, WINDOW)],
                o_vmem.at[pl.ds(EMBED_DIM // 8, GAZ_DIM // 8), 0], zsem)
            gcp = pltpu.async_copy(table_hbm.at[i_vmem.at[0]], scr, gsem)
            gcp.wait()

            tok = lax.iota(jnp.int32, LANES)

            @pl.loop(0, EMBED_DIM // 8)
            def _(jb):
                for jr in range(8):
                    f = jb * 8 + jr
                    fvec = jnp.full((LANES,), f, jnp.int32)
                    for k in range(WINDOW // LANES):
                        vals = plsc.load_gather(scr, [tok + k * LANES, fvec])
                        o_vmem[jb, 0, jr, pl.ds(k * LANES, LANES)] = vals

            zcp.wait()

        pltpu.emit_pipeline(
            body,
            grid=(nw,),
            in_specs=[
                pl.BlockSpec((1, WINDOW), lambda i: (i, 0)),
            ],
            out_specs=[
                pl.BlockSpec((OUT_DIM // 8, 1, 8, WINDOW),
                             lambda i: (0, i, 0, 0)),
            ],
            core_axis_name=("core", "subcore"),
            dimension_semantics=(pltpu.PARALLEL,),
            _explicit_indices=True,
        )(idx_hbm, out_hbm)

    out_tiled = kern(idx2d, gaz_t3, word_table)
    return out_tiled.transpose(1, 3, 0, 2).reshape(num_tokens, OUT_DIM)


def kernel(sentence_data, batch_sizes, gazetteers_data, word_table):
    out = _embed_concat(sentence_data, gazetteers_data, word_table)
    return out, batch_sizes
